# Initial kernel scaffold; baseline (speedup 1.0000x reference)
import jax
import jax.numpy as jnp
from jax.experimental import pallas as pl


def kernel(sent_list, table):
    raise NotImplementedError("write your pallas kernel here")


# speedup vs baseline: 1.0226x; 1.0226x over previous
import functools

import jax
import jax.numpy as jnp
from jax import lax
from jax.experimental import pallas as pl
from jax.experimental.pallas import tpu as pltpu
from jax.experimental.pallas import tpu_sc as plsc

EMB_DIM = 32
BATCH = 16384
HIST = 50
B = BATCH * HIST

NC, NS = 2, 16
NW = NC * NS
B_PER_W = B // NW
CH = 128
N_CH = B_PER_W // CH

_mesh = plsc.VectorSubcoreMesh(core_axis_name="c", subcore_axis_name="s")


@functools.partial(
    pl.kernel,
    mesh=_mesh,
    out_type=jax.ShapeDtypeStruct((B, EMB_DIM), jnp.float32),
    compiler_params=pltpu.CompilerParams(use_tc_tiling_on_sc=False),
    scratch_types=[
        pltpu.VMEM((B_PER_W,), jnp.int32),
        pltpu.VMEM((CH, EMB_DIM), jnp.float32),
        pltpu.SemaphoreType.DMA,
    ],
)
def _embed_gather(idx_hbm, table_hbm, out_hbm, idx_v, buf, sem):
    wid = lax.axis_index("s") * NC + lax.axis_index("c")
    base = wid * B_PER_W
    pltpu.sync_copy(idx_hbm.at[pl.ds(base, B_PER_W)], idx_v)

    def body(j, carry):
        pltpu.async_copy(
            table_hbm.at[idx_v.at[pl.ds(j * CH, CH)]], buf, sem
        ).wait()
        pltpu.sync_copy(buf, out_hbm.at[pl.ds(base + j * CH, CH)])
        return carry

    lax.fori_loop(0, N_CH, body, 0)


def kernel(sent_list, table):
    idx = sent_list.reshape(-1)
    out = _embed_gather(idx, table)
    return out.reshape(BATCH, HIST, EMB_DIM)


# speedup vs baseline: 1.1071x; 1.0827x over previous
import functools

import jax
import jax.numpy as jnp
from jax import lax
from jax.experimental import pallas as pl
from jax.experimental.pallas import tpu as pltpu
from jax.experimental.pallas import tpu_sc as plsc

EMB_DIM = 32
BATCH = 16384
HIST = 50
B = BATCH * HIST

NC, NS = 2, 16
NW = NC * NS
B_PER_W = B // NW
CH = 128
N_CH = B_PER_W // CH
NBUF = 4
NG = N_CH // NBUF

_mesh = plsc.VectorSubcoreMesh(core_axis_name="c", subcore_axis_name="s")


@functools.partial(
    pl.kernel,
    mesh=_mesh,
    out_type=jax.ShapeDtypeStruct((B, EMB_DIM), jnp.float32),
    compiler_params=pltpu.CompilerParams(use_tc_tiling_on_sc=False),
    scratch_types=[
        pltpu.VMEM((B_PER_W,), jnp.int32),
        pltpu.VMEM((NBUF, CH, EMB_DIM), jnp.float32),
        pltpu.SemaphoreType.DMA((NBUF,)),
        pltpu.SemaphoreType.DMA((NBUF,)),
    ],
)
def _embed_gather(idx_hbm, table_hbm, out_hbm, idx_v, bufs, sem_g, sem_o):
    wid = lax.axis_index("s") * NC + lax.axis_index("c")
    base = wid * B_PER_W
    pltpu.sync_copy(idx_hbm.at[pl.ds(base, B_PER_W)], idx_v)

    def start_gather(j, b):
        pltpu.async_copy(
            table_hbm.at[idx_v.at[pl.ds(j * CH, CH)]], bufs.at[b], sem_g.at[b]
        )

    def wait_gather(b):
        pltpu.make_async_copy(table_hbm.at[idx_v.at[pl.ds(0, CH)]],
                              bufs.at[b], sem_g.at[b]).wait()

    def start_outcopy(j, b):
        pltpu.async_copy(bufs.at[b], out_hbm.at[pl.ds(base + j * CH, CH)],
                         sem_o.at[b])

    def wait_outcopy(b):
        pltpu.make_async_copy(bufs.at[b], out_hbm.at[pl.ds(0, CH)],
                              sem_o.at[b]).wait()

    for b in range(3):
        start_gather(b, b)

    for b in range(NBUF):
        j = b
        wait_gather(b)
        start_outcopy(j, b)
        nb = (b + 3) % NBUF
        if b >= 1:
            wait_outcopy(nb)
        start_gather(j + 3, nb)

    def body(g, carry):
        for b in range(NBUF):
            j = g * NBUF + b
            wait_gather(b)
            start_outcopy(j, b)
            nb = (b + 3) % NBUF
            wait_outcopy(nb)
            start_gather(j + 3, nb)
        return carry

    lax.fori_loop(1, NG - 1, body, 0)

    g = NG - 1
    for b in range(NBUF):
        j = g * NBUF + b
        wait_gather(b)
        start_outcopy(j, b)
        nb = (b + 3) % NBUF
        wait_outcopy(nb)
        if b == 0:
            start_gather(j + 3, nb)
    wait_outcopy((NBUF - 1) % NBUF)


def kernel(sent_list, table):
    idx = sent_list.reshape(-1)
    out = _embed_gather(idx, table)
    return out.reshape(BATCH, HIST, EMB_DIM)


# speedup vs baseline: 1.6457x; 1.4865x over previous
import functools

import jax
import jax.numpy as jnp
from jax import lax
from jax.experimental import pallas as pl
from jax.experimental.pallas import tpu as pltpu
from jax.experimental.pallas import tpu_sc as plsc

EMB_DIM = 32
BATCH = 16384
HIST = 50

NC, NS = 2, 16
NW = NC * NS
COLS_PER_W = BATCH // NW
NBT = COLS_PER_W // 128
UNITS = HIST * NBT
NBUF = 4
NG = UNITS // NBUF

_mesh = plsc.VectorSubcoreMesh(core_axis_name="c", subcore_axis_name="s")


@functools.partial(
    pl.kernel,
    mesh=_mesh,
    out_type=jax.ShapeDtypeStruct((HIST, EMB_DIM // 8, BATCH // 128, 8, 128),
                                  jnp.float32),
    compiler_params=pltpu.CompilerParams(use_tc_tiling_on_sc=False, needs_layout_passes=False),
    scratch_types=[
        pltpu.VMEM((HIST, COLS_PER_W), jnp.int32),
        pltpu.VMEM((NBUF, 128, EMB_DIM), jnp.float32),
        pltpu.VMEM((NBUF, 1, EMB_DIM // 8, 1, 8, 128), jnp.float32),
        pltpu.SemaphoreType.DMA((NBUF,)),
        pltpu.SemaphoreType.DMA((NBUF,)),
    ],
)
def _embed_gather(idx_hbm, table_hbm, out_hbm, idx_v, gbufs, tbufs,
                  sem_g, sem_o):
    wid = lax.axis_index("s") * NC + lax.axis_index("c")
    col0 = wid * COLS_PER_W
    pltpu.sync_copy(idx_hbm.at[:, pl.ds(col0, COLS_PER_W)], idx_v)

    iota16 = lax.iota(jnp.int32, 16)
    zeros16 = jnp.zeros((16,), jnp.int32)

    def start_gather(j, b):
        h = j // NBT
        bt = j % NBT
        pltpu.async_copy(
            table_hbm.at[idx_v.at[h, pl.ds(bt * 128, 128)]],
            gbufs.at[b], sem_g.at[b],
        )

    def wait_gather(b):
        pltpu.make_async_copy(
            table_hbm.at[idx_v.at[0, pl.ds(0, 128)]],
            gbufs.at[b], sem_g.at[b],
        ).wait()

    def transpose(b):
        def body_k(k, carry):
            rows = iota16 + k * 16
            for d in range(EMB_DIM):
                v = plsc.load_gather(
                    gbufs.at[b], [rows, zeros16 + d])
                tbufs[b, 0, d // 8, 0, d % 8, pl.ds(k * 16, 16)] = v
            return carry

        lax.fori_loop(0, 128 // 16, body_k, 0)

    def start_outcopy(j, b):
        h = j // NBT
        bt = j % NBT
        pltpu.async_copy(
            tbufs.at[b],
            out_hbm.at[pl.ds(h, 1), pl.ds(0, EMB_DIM // 8),
                       pl.ds(wid * NBT + bt, 1), pl.ds(0, 8), pl.ds(0, 128)],
            sem_o.at[b],
        )

    def wait_outcopy(b):
        pltpu.make_async_copy(
            tbufs.at[b],
            out_hbm.at[pl.ds(0, 1), pl.ds(0, EMB_DIM // 8),
                       pl.ds(0, 1), pl.ds(0, 8), pl.ds(0, 128)],
            sem_o.at[b],
        ).wait()

    for b in range(NBUF):
        start_gather(b, b)

    for b in range(NBUF):
        j = b
        wait_gather(b)
        transpose(b)
        start_outcopy(j, b)
        start_gather(j + NBUF, b)

    def body(g, carry):
        for b in range(NBUF):
            j = g * NBUF + b
            wait_gather(b)
            wait_outcopy(b)
            transpose(b)
            start_outcopy(j, b)
            start_gather(j + NBUF, b)
        return carry

    lax.fori_loop(1, NG - 1, body, 0)

    g = NG - 1
    for b in range(NBUF):
        j = g * NBUF + b
        wait_gather(b)
        wait_outcopy(b)
        transpose(b)
        start_outcopy(j, b)
    for b in range(NBUF):
        wait_outcopy(b)


def kernel(sent_list, table):
    idx_lin = jnp.abs(sent_list.T)
    out5d = _embed_gather(idx_lin, table)
    return out5d.transpose((2, 4, 0, 1, 3)).reshape(BATCH, HIST, EMB_DIM)
